# baseline (speedup 1.0000x reference)
import jax
import jax.numpy as jnp
from jax.experimental import pallas as pl
from jax.experimental.pallas import tpu as pltpu

Q = 1024
D = 384
BK = 4000
RCH = 8


def _knn_body(q_ref, v_ref, idx_out, val_out, qn_ref, best_ref, bidx_ref,
              sims_ref):
    j = pl.program_id(0)
    nb = pl.num_programs(0)

    @pl.when(j == 0)
    def _init():
        q = q_ref[...]
        qnorm = jnp.sqrt(jnp.sum(q * q, axis=1, keepdims=True))
        qn_ref[...] = (q / jnp.maximum(qnorm, 1e-12)).astype(jnp.bfloat16)
        best_ref[...] = jnp.full((1, Q), -jnp.inf, jnp.float32)
        bidx_ref[...] = jnp.zeros((1, Q), jnp.int32)

    v = v_ref[...]
    vnorm = jnp.sqrt(jnp.sum(v * v, axis=1, keepdims=True))
    vn = v / jnp.maximum(vnorm, 1e-12)
    sims_ref[...] = jax.lax.dot_general(
        vn.astype(jnp.bfloat16), qn_ref[...],
        (((1,), (1,)), ((), ())),
        preferred_element_type=jnp.float32)

    def step(r, carry):
        run_m, run_r = carry
        chunk = sims_ref[pl.ds(r * RCH, RCH), :]
        upd = chunk > run_m
        run_m = jnp.where(upd, chunk, run_m)
        run_r = jnp.where(upd, jnp.broadcast_to(r, (RCH, Q)).astype(jnp.int32),
                          run_r)
        return run_m, run_r

    init_m = jnp.full((RCH, Q), -jnp.inf, jnp.float32)
    init_r = jnp.zeros((RCH, Q), jnp.int32)
    run_m, run_r = jax.lax.fori_loop(0, BK // RCH, step, (init_m, init_r),
                                     unroll=8)
    sub = jax.lax.broadcasted_iota(jnp.int32, (RCH, Q), 0)
    cand_i = run_r * RCH + sub
    bmax = jnp.max(run_m, axis=0)[None, :]
    cand = jnp.where(run_m == bmax, cand_i, BK)
    barg = jnp.min(cand, axis=0)[None, :]

    upd = bmax > best_ref[...]
    bidx_ref[...] = jnp.where(upd, barg + j * BK, bidx_ref[...])
    best_ref[...] = jnp.where(upd, bmax, best_ref[...])

    @pl.when(j == nb - 1)
    def _fin():
        idx_out[...] = bidx_ref[...]
        val_out[...] = best_ref[...]


def kernel(v_labels, vectors):
    k = vectors.shape[0]
    nb = k // BK
    idx, val = pl.pallas_call(
        _knn_body,
        grid=(nb,),
        in_specs=[
            pl.BlockSpec((Q, D), lambda j: (0, 0)),
            pl.BlockSpec((BK, D), lambda j: (j, 0)),
        ],
        out_specs=[
            pl.BlockSpec((1, Q), lambda j: (0, 0)),
            pl.BlockSpec((1, Q), lambda j: (0, 0)),
        ],
        out_shape=[
            jax.ShapeDtypeStruct((1, Q), jnp.int32),
            jax.ShapeDtypeStruct((1, Q), jnp.float32),
        ],
        scratch_shapes=[
            pltpu.VMEM((Q, D), jnp.bfloat16),
            pltpu.VMEM((1, Q), jnp.float32),
            pltpu.VMEM((1, Q), jnp.int32),
            pltpu.VMEM((BK, Q), jnp.float32),
        ],
    )(v_labels, vectors)
    return idx.reshape(Q), val.reshape(Q)


# speedup vs baseline: 1.3507x; 1.3507x over previous
import jax
import jax.numpy as jnp
from jax.experimental import pallas as pl
from jax.experimental.pallas import tpu as pltpu

Q = 1024
D = 384
BK = 10000
RC = 10
CH = BK // RC
G = 8


def _knn_body(q_ref, v_ref, idx_out, val_out, qn_ref, best_ref, bidx_ref):
    j = pl.program_id(0)
    nb = pl.num_programs(0)

    @pl.when(j == 0)
    def _init():
        q = q_ref[...]
        qnorm = jnp.sqrt(jnp.sum(q * q, axis=1, keepdims=True))
        qn_ref[...] = (q / jnp.maximum(qnorm, 1e-12)).astype(jnp.bfloat16)
        best_ref[...] = jnp.full((1, Q), -jnp.inf, jnp.float32)
        bidx_ref[...] = jnp.zeros((1, Q), jnp.int32)

    v = v_ref[...]
    vnorm = jnp.sqrt(jnp.sum(v * v, axis=1, keepdims=True))
    vn = (v / jnp.maximum(vnorm, 1e-12)).astype(jnp.bfloat16)
    qn = qn_ref[...]

    for t in range(RC):
        vc = jax.lax.slice(vn, (t * CH, 0), ((t + 1) * CH, D))
        sims = jax.lax.dot_general(
            vc, qn, (((1,), (1,)), ((), ())),
            preferred_element_type=jnp.float32)
        sims3 = sims.reshape(CH // G, G, Q)
        m8 = jnp.max(sims3, axis=0)
        a8 = jnp.argmax(sims3, axis=0)
        sub = jax.lax.broadcasted_iota(jnp.int32, (G, Q), 0)
        cand_i = a8.astype(jnp.int32) * G + sub
        bmax = jnp.max(m8, axis=0)[None, :]
        cand = jnp.where(m8 == bmax, cand_i, CH)
        barg = jnp.min(cand, axis=0)[None, :]

        upd = bmax > best_ref[...]
        bidx_ref[...] = jnp.where(upd, barg + j * BK + t * CH, bidx_ref[...])
        best_ref[...] = jnp.where(upd, bmax, best_ref[...])

    @pl.when(j == nb - 1)
    def _fin():
        idx_out[...] = bidx_ref[...]
        val_out[...] = best_ref[...]


def kernel(v_labels, vectors):
    k = vectors.shape[0]
    nb = k // BK
    idx, val = pl.pallas_call(
        _knn_body,
        grid=(nb,),
        in_specs=[
            pl.BlockSpec((Q, D), lambda j: (0, 0)),
            pl.BlockSpec((BK, D), lambda j: (j, 0)),
        ],
        out_specs=[
            pl.BlockSpec((1, Q), lambda j: (0, 0)),
            pl.BlockSpec((1, Q), lambda j: (0, 0)),
        ],
        out_shape=[
            jax.ShapeDtypeStruct((1, Q), jnp.int32),
            jax.ShapeDtypeStruct((1, Q), jnp.float32),
        ],
        scratch_shapes=[
            pltpu.VMEM((Q, D), jnp.bfloat16),
            pltpu.VMEM((1, Q), jnp.float32),
            pltpu.VMEM((1, Q), jnp.int32),
        ],
    )(v_labels, vectors)
    return idx.reshape(Q), val.reshape(Q)
